# baseline (speedup 1.0000x reference)
import functools

import jax
import jax.numpy as jnp
from jax import lax
from jax.experimental import pallas as pl
from jax.experimental.pallas import tpu as pltpu
from jax.experimental.pallas import tpu_sc as plsc

HID = 1024
EPS = 1e-5
POS_OFF = 2
NC, NS = 2, 16
NW = NC * NS
CHUNK = 32
NSPLIT = 2


def _gather_call(ids4, table, part, npart, nch):
    mesh = plsc.VectorSubcoreMesh(core_axis_name="c", subcore_axis_name="s")

    @functools.partial(
        pl.kernel,
        mesh=mesh,
        out_type=jax.ShapeDtypeStruct((npart, HID), jnp.float32),
        scratch_types=[
            pltpu.VMEM((nch, CHUNK), jnp.int32),
            pltpu.VMEM((CHUNK, HID), jnp.float32),
            pltpu.VMEM((CHUNK, HID), jnp.float32),
            pltpu.SemaphoreType.DMA,
            pltpu.SemaphoreType.DMA,
        ],
    )
    def k(ids_hbm, table_hbm, out_hbm, idx_v, buf0, buf1, sem0, sem1):
        wid = lax.axis_index("s") * NC + lax.axis_index("c")
        base = wid * (nch * CHUNK)
        pltpu.sync_copy(ids_hbm.at[part, wid], idx_v)
        bufs = (buf0, buf1)
        sems = (sem0, sem1)
        copies = [None, None]
        copies[0] = pltpu.async_copy(table_hbm.at[idx_v.at[0]], bufs[0], sems[0])
        for c in range(nch):
            cur = c & 1
            nxt = (c + 1) & 1
            if c + 1 < nch:
                copies[nxt] = pltpu.async_copy(
                    table_hbm.at[idx_v.at[c + 1]], bufs[nxt], sems[nxt])
            copies[cur].wait()
            pltpu.sync_copy(bufs[cur], out_hbm.at[pl.ds(base + c * CHUNK, CHUNK)])

    return k(ids4, table)


def _ln_body(x_ref, p_ref, g_ref, b_ref, o_ref):
    x = x_ref[...] + p_ref[...]
    mean = jnp.mean(x, axis=-1, keepdims=True)
    xc = x - mean
    var = jnp.mean(xc * xc, axis=-1, keepdims=True)
    o_ref[...] = xc * lax.rsqrt(var + EPS) * g_ref[...] + b_ref[...]


def _ln_body_first(x_ref, p_ref, g_ref, b_ref, o_ref):
    _ln_body(x_ref, p_ref, g_ref, b_ref, o_ref)


def _ln_body_rest(x_ref, acc_ref, p_ref, g_ref, b_ref, o_ref):
    del acc_ref
    _ln_body(x_ref, p_ref, g_ref, b_ref, o_ref)


def kernel(input_ids, W, P, gamma, beta):
    B, S = input_ids.shape
    n = B * S
    sp = S // NSPLIT
    npart = n // NSPLIT
    nch = npart // (NW * CHUNK)
    ids4 = (input_ids.astype(jnp.int32)
            .reshape(B, NSPLIT, sp)
            .transpose(1, 0, 2)
            .reshape(NSPLIT, NW, nch, CHUNK))
    p_slice = lax.slice(P, (POS_OFF, 0), (POS_OFF + S, HID))
    g2 = gamma.reshape(1, HID)
    b2 = beta.reshape(1, HID)
    parts = [_gather_call(ids4, W, k, npart, nch) for k in range(NSPLIT)]

    x_spec = pl.BlockSpec((sp, HID), lambda b: (b, 0))
    v_spec = pl.BlockSpec((1, HID), lambda b: (0, 0))
    out_shape = jax.ShapeDtypeStruct((n, HID), jnp.float32)

    def p_spec(k):
        return pl.BlockSpec((sp, HID), lambda b: (k, 0))

    def out_spec(k):
        return pl.BlockSpec((sp, HID), lambda b: (b * NSPLIT + k, 0))

    acc = pl.pallas_call(
        _ln_body_first,
        grid=(B,),
        in_specs=[x_spec, p_spec(0), v_spec, v_spec],
        out_specs=out_spec(0),
        out_shape=out_shape,
    )(parts[0], p_slice, g2, b2)
    for k in range(1, NSPLIT):
        acc = pl.pallas_call(
            _ln_body_rest,
            grid=(B,),
            in_specs=[
                x_spec,
                pl.BlockSpec(memory_space=pltpu.HBM),
                p_spec(k), v_spec, v_spec,
            ],
            out_specs=out_spec(k),
            out_shape=out_shape,
            input_output_aliases={1: 0},
        )(parts[k], acc, p_slice, g2, b2)
    return acc.reshape(B, S, HID)


# speedup vs baseline: 1.0356x; 1.0356x over previous
import functools

import jax
import jax.numpy as jnp
from jax import lax
from jax.experimental import pallas as pl
from jax.experimental.pallas import tpu as pltpu
from jax.experimental.pallas import tpu_sc as plsc

HID = 1024
EPS = 1e-5
POS_OFF = 2
NC, NS = 2, 16
NW = NC * NS
CHUNK = 32
NSPLIT = 1
BS = 2048


def _gather_call(ids4, table, part, npart, nch):
    mesh = plsc.VectorSubcoreMesh(core_axis_name="c", subcore_axis_name="s")

    @functools.partial(
        pl.kernel,
        mesh=mesh,
        out_type=jax.ShapeDtypeStruct((npart, HID), jnp.float32),
        scratch_types=[
            pltpu.VMEM((nch, CHUNK), jnp.int32),
            pltpu.VMEM((CHUNK, HID), jnp.float32),
            pltpu.VMEM((CHUNK, HID), jnp.float32),
            pltpu.VMEM((CHUNK, HID), jnp.float32),
            pltpu.SemaphoreType.DMA,
            pltpu.SemaphoreType.DMA,
            pltpu.SemaphoreType.DMA,
        ],
    )
    def k(ids_hbm, table_hbm, out_hbm, idx_v, buf0, buf1, buf2,
          sem0, sem1, sem2):
        wid = lax.axis_index("s") * NC + lax.axis_index("c")
        base = wid * (nch * CHUNK)
        pltpu.sync_copy(ids_hbm.at[part, wid], idx_v)
        bufs = (buf0, buf1, buf2)
        sems = (sem0, sem1, sem2)
        nbuf = 3
        copies = [None] * nbuf

        def issue(j):
            return pltpu.async_copy(table_hbm.at[idx_v.at[j]],
                                    bufs[j % nbuf], sems[j % nbuf])

        for j in range(min(nbuf - 1, nch)):
            copies[j % nbuf] = issue(j)
        for c in range(nch):
            nxt = c + nbuf - 1
            if nxt < nch:
                copies[nxt % nbuf] = issue(nxt)
            copies[c % nbuf].wait()
            pltpu.sync_copy(bufs[c % nbuf],
                            out_hbm.at[pl.ds(base + c * CHUNK, CHUNK)])

    return k(ids4, table)


def _ln_body(x_ref, p_ref, g_ref, b_ref, o_ref):
    x = x_ref[...] + p_ref[...]
    mean = jnp.mean(x, axis=-1, keepdims=True)
    xc = x - mean
    var = jnp.mean(xc * xc, axis=-1, keepdims=True)
    o_ref[...] = xc * lax.rsqrt(var + EPS) * g_ref[...] + b_ref[...]


def _ln_body_first(x_ref, p_ref, g_ref, b_ref, o_ref):
    _ln_body(x_ref, p_ref, g_ref, b_ref, o_ref)


def _ln_body_rest(x_ref, acc_ref, p_ref, g_ref, b_ref, o_ref):
    del acc_ref
    _ln_body(x_ref, p_ref, g_ref, b_ref, o_ref)


def kernel(input_ids, W, P, gamma, beta):
    B, S = input_ids.shape
    n = B * S
    npart = n // NSPLIT
    nch = npart // (NW * CHUNK)
    ids4 = input_ids.astype(jnp.int32).reshape(NSPLIT, NW, nch, CHUNK)
    p_slice = lax.slice(P, (POS_OFF, 0), (POS_OFF + S, HID))
    g2 = gamma.reshape(1, HID)
    b2 = beta.reshape(1, HID)
    parts = [_gather_call(ids4, W, k, npart, nch) for k in range(NSPLIT)]

    nsb = S // BS
    bp = B // NSPLIT
    x_spec = pl.BlockSpec((BS, HID), lambda s, b: (b * nsb + s, 0))
    p_spec = pl.BlockSpec((BS, HID), lambda s, b: (s, 0))
    v_spec = pl.BlockSpec((1, HID), lambda s, b: (0, 0))
    out_shape = jax.ShapeDtypeStruct((n, HID), jnp.float32)

    def out_spec(k):
        return pl.BlockSpec((BS, HID), lambda s, b: ((k * bp + b) * nsb + s, 0))

    acc = pl.pallas_call(
        _ln_body_first,
        grid=(nsb, bp),
        in_specs=[x_spec, p_spec, v_spec, v_spec],
        out_specs=out_spec(0),
        out_shape=out_shape,
    )(parts[0], p_slice, g2, b2)
    for k in range(1, NSPLIT):
        acc = pl.pallas_call(
            _ln_body_rest,
            grid=(nsb, bp),
            in_specs=[
                x_spec,
                pl.BlockSpec(memory_space=pltpu.HBM),
                p_spec, v_spec, v_spec,
            ],
            out_specs=out_spec(k),
            out_shape=out_shape,
            input_output_aliases={1: 0},
        )(parts[k], acc, p_slice, g2, b2)
    return acc.reshape(B, S, HID)
